# baseline (speedup 1.0000x reference)
import jax
import jax.numpy as jnp
from jax.experimental import pallas as pl

N = 10000
E = 320000
E2 = E + N
DELTA = 2.0
K_KEEP = 5000
MAXDEG = 96

_B_TAIL = {
    32: [269184, 289536, 309888],
    16: [269136, 289440, 309744],
    8: [269136, 289440, 309744],
}


def _boundaries(d):
    return [20736 * k for k in range(1, 13)] + _B_TAIL[d]


def _matmul_kernel(x_ref, w_ref, o_ref):
    o_ref[...] = jnp.dot(x_ref[...], w_ref[...], preferred_element_type=jnp.float32)


def _emb_matmul(x, w, b):
    out = pl.pallas_call(
        _matmul_kernel,
        grid=(10,),
        in_specs=[
            pl.BlockSpec((1000, 128), lambda i: (i, 0)),
            pl.BlockSpec((128, 128), lambda i: (0, 0)),
        ],
        out_specs=pl.BlockSpec((1000, 128), lambda i: (i, 0)),
        out_shape=jax.ShapeDtypeStruct((N, 128), jnp.float32),
    )(x, w)
    return out + b


def _seg_fold(hw, src_pad, ew_pad, g_t, ds, starts, d):
    nsteps = g_t.shape[0]

    def body(j, acc):
        idx = g_t[j]
        w = ew_pad[idx]
        s = src_pad[idx]
        return acc + w[:, None] * hw[s]

    out = jax.lax.fori_loop(0, nsteps, body, jnp.zeros((N, d), jnp.float32))

    for b in _boundaries(d):
        r = ds[b]
        valid = ds[b - 1] == r
        c = b - starts[r]
        grow = g_t[:, r]

        def bodyc(j, carry):
            left, right = carry
            idx = grow[j]
            term = ew_pad[idx] * hw[s_pad_row(idx)]
            lmask = j < c
            left = left + jnp.where(lmask, term, 0.0)
            right = right + jnp.where(lmask, 0.0, term)
            return (left, right)

        def s_pad_row(idx):
            return src_pad[idx]

        left, right = jax.lax.fori_loop(
            0, nsteps, bodyc,
            (jnp.zeros((d,), jnp.float32), jnp.zeros((d,), jnp.float32)))
        fixed = left + right
        out = out.at[r].set(jnp.where(valid, fixed, out[r]))
    return out


def kernel(x, edge_index, edge_weight, emb_w, emb_b, gcn_ws, gcn_bs,
           mlp_ws, mlp_bs, final_w, final_b):
    row, col = edge_index[0], edge_index[1]

    x_emb = _emb_matmul(x, emb_w, emb_b)

    deg = jax.ops.segment_sum(edge_weight, row, num_segments=N)
    dinv = jnp.where(deg > 0, jax.lax.rsqrt(jnp.maximum(deg, 1e-12)), 0.0)
    w_norm = dinv[row] * edge_weight * dinv[col]

    loops = jnp.arange(N, dtype=edge_index.dtype)
    src_full = jnp.concatenate([row, loops])
    dst_full = jnp.concatenate([col, loops])
    ew_full = jnp.concatenate(
        [DELTA * w_norm, (1.0 - DELTA) * jnp.ones((N,), jnp.float32)])

    perm = jnp.argsort(dst_full, stable=True)
    ds = dst_full[perm]
    starts = jnp.searchsorted(ds, jnp.arange(N, dtype=jnp.int32))
    occ = jnp.arange(E2, dtype=jnp.int32) - starts[ds].astype(jnp.int32)
    g = jnp.full((N, MAXDEG), E2, jnp.int32)
    g = g.at[ds, occ].set(perm.astype(jnp.int32), unique_indices=True)
    g_t = g.T

    src_pad = jnp.concatenate([src_full, jnp.zeros((1,), src_full.dtype)])
    ew_pad = jnp.concatenate([ew_full, jnp.zeros((1,), jnp.float32)])

    h = x_emb
    for w, b in zip(gcn_ws, gcn_bs):
        hw = h @ w
        d = w.shape[1]
        out = _seg_fold(hw, src_pad, ew_pad, g_t, ds, starts, d)
        h = jnp.tanh(out + b)
    for w, b in zip(mlp_ws, mlp_bs):
        h = jax.nn.relu(h @ w + b)
    score = jnp.tanh(h @ final_w + final_b)

    zs = score[:, 0]
    num = jnp.sum(edge_weight * zs[row] * zs[col])
    den = jnp.sum(edge_weight) + 1e-12
    aux_loss = num / den

    _, kept_nodes = jax.lax.top_k(zs, K_KEEP)
    return x_emb, kept_nodes, zs, aux_loss


# speedup vs baseline: 27.8997x; 27.8997x over previous
import functools
import jax
import jax.numpy as jnp
from jax import lax
from jax.experimental import pallas as pl
from jax.experimental.pallas import tpu as pltpu, tpu_sc as plsc

N = 10000
E = 320000
E2 = E + N
DELTA = 2.0
K_KEEP = 5000

WSZ = 384
OV = 128
ROWS_CAP = 1024
NW = 32
NDUMP = 64
NSIDE = 16
NOUT = N + NDUMP + NSIDE

_B_TAIL = {32: [269184, 289536, 309888], 16: [269136, 289440, 309744]}


def _boundaries(dpad):
    return [20736 * k for k in range(1, 13)] + _B_TAIL[dpad]


def _chunk_tables(dpad):
    bf = [0] + _boundaries(dpad) + [E2]
    plo, phi, mids = [], [], []
    for c in range(16):
        lo, hi = bf[c], bf[c + 1]
        mid = lo + (hi - lo) // 2
        mids.append(mid)
        plo += [lo, mid - OV]
        phi += [mid, hi]
    sbase = [max(0, (N * bf[c]) // E2 - 64) for c in range(16)]
    return bf, plo, phi, mids, sbase


def _prep(dst_full, src_full, ew_full, dpad):
    bf, _, _, mids, sbase = _chunk_tables(dpad)
    bnds = jnp.asarray(bf[1:-1], jnp.int32)
    perm = jnp.argsort(dst_full, stable=True).astype(jnp.int32)
    ds = dst_full[perm].astype(jnp.int32)
    srcs = src_full[perm].astype(jnp.int32)
    ews = ew_full[perm].astype(jnp.float32)
    chg = jnp.concatenate([jnp.ones((1,), bool), ds[1:] != ds[:-1]])
    keep = jnp.where(chg, 0.0, 1.0).astype(jnp.float32)
    chunk_of = jnp.searchsorted(bnds, jnp.arange(E2, dtype=jnp.int32),
                                side="right")
    sbase_arr = jnp.asarray(sbase, jnp.int32)
    locs = ds - sbase_arr[chunk_of]
    own_rows = []
    rfix = []
    valid = []
    slot_ids = jnp.arange(ROWS_CAP, dtype=jnp.int32)
    dump = (N + (slot_ids % NDUMP)).astype(jnp.int32)
    for c in range(16):
        lo, hi, mid, sb = bf[c], bf[c + 1], mids[c], sbase[c]
        r_lo, r_mid, r_midp, r_hi = ds[lo], ds[mid], ds[mid - 1], ds[hi - 1]
        slot_rows = sb + slot_ids
        top0 = r_midp - (r_midp == r_mid)
        d0 = jnp.where((slot_rows >= r_lo) & (slot_rows <= top0),
                       slot_rows, dump)
        own_rows.append(d0)
        if c < 15:
            strad = ds[hi - 1] == ds[hi]
            rfix.append(ds[hi])
            valid.append(strad.astype(jnp.float32))
        else:
            strad = jnp.zeros((), bool)
        top1 = r_hi - strad
        d1 = jnp.where((slot_rows >= r_mid) & (slot_rows <= top1),
                       slot_rows, dump)
        d1 = jnp.where((slot_rows == r_hi) & strad,
                       jnp.int32(N + NDUMP + c), d1)
        own_rows.append(d1)
    own_dst = jnp.stack(own_rows).reshape(NW, ROWS_CAP // 128, 128)
    pad_i = jnp.zeros((WSZ + 8,), jnp.int32)
    pad_f = jnp.zeros((WSZ + 8,), jnp.float32)
    return dict(
        srcs=jnp.concatenate([srcs, pad_i]),
        ews=jnp.concatenate([ews, pad_f]),
        keep=jnp.concatenate([keep, pad_f]),
        locs=jnp.concatenate([locs, pad_i]),
        own_dst=own_dst,
        rfix=jnp.stack(rfix + [jnp.int32(-1)]),
        valid=jnp.stack(valid + [jnp.float32(0.0)]))


def _psel(w, table):
    v = jnp.int32(table[NW - 1])
    for i in range(NW - 2, -1, -1):
        v = jnp.where(w == i, jnp.int32(table[i]), v)
    return v


def _fold_body(nv, plo_t, phi_t,
               hw, srcs, ews, keepf, locs, own_dst,
               out,
               idx_v, rows_v, out_local, ew_v, kp_v, lc_v, dst_v,
               sem_g, sem_s):
    c = lax.axis_index("c")
    s = lax.axis_index("s")
    w = s * 2 + c
    plo = _psel(w, plo_t)
    phi = _psel(w, phi_t)
    skip = lax.rem(plo, 8)
    abase = plo - skip
    ne = phi - plo
    nwin = lax.div(skip + ne + WSZ - 1, WSZ)

    zero = jnp.zeros((16,), jnp.float32)
    cols = [lax.broadcasted_iota(jnp.int32, (16,), 0) + t * 16
            for t in range(nv)]

    def window(g, acc):
        base = pl.multiple_of(abase + g * WSZ, 8)
        pltpu.sync_copy(srcs.at[pl.ds(base, WSZ)], idx_v)
        pltpu.async_copy(hw.at[idx_v], rows_v, sem_g).wait()
        pltpu.sync_copy(ews.at[pl.ds(base, WSZ)], ew_v)
        pltpu.sync_copy(keepf.at[pl.ds(base, WSZ)], kp_v)
        pltpu.sync_copy(locs.at[pl.ds(base, WSZ)], lc_v)
        j_lo = jnp.where(g == 0, skip, 0)
        j_hi = jnp.minimum(WSZ, skip + ne - g * WSZ)

        def body(j, a):
            jv = jnp.full((16,), j, jnp.int32)
            vew = plsc.load_gather(ew_v, [jv])
            vkp = plsc.load_gather(kp_v, [jv])
            vlc = plsc.load_gather(lc_v, [jv])
            na = []
            for t in range(nv):
                rowt = rows_v[j, pl.ds(t * 16, 16)]
                at = a[t] * vkp + vew * rowt
                plsc.store_scatter(out_local, [vlc, cols[t]], at)
                na.append(at)
            return tuple(na)

        return lax.fori_loop(j_lo, j_hi, body, acc)

    lax.fori_loop(0, nwin, window, (zero,) * nv)

    pltpu.sync_copy(own_dst.at[w], dst_v)
    for b in range(ROWS_CAP // 128):
        pltpu.async_copy(out_local.at[pl.ds(b * 128, 128)],
                         out.at[dst_v.at[b]], sem_s).wait()


@functools.cache
def _make_fold(dpad):
    nv = dpad // 16
    _, plo_t, phi_t, _, _ = _chunk_tables(dpad)
    mesh = plsc.VectorSubcoreMesh(core_axis_name="c", subcore_axis_name="s")
    return functools.partial(
        pl.kernel,
        mesh=mesh,
        compiler_params=pltpu.CompilerParams(use_tc_tiling_on_sc=False,
                                             needs_layout_passes=False),
        out_type=[jax.ShapeDtypeStruct((NOUT, dpad), jnp.float32)],
        scratch_types=[
            pltpu.VMEM((WSZ,), jnp.int32),
            pltpu.VMEM((WSZ, dpad), jnp.float32),
            pltpu.VMEM((ROWS_CAP, dpad), jnp.float32),
            pltpu.VMEM((WSZ,), jnp.float32),
            pltpu.VMEM((WSZ,), jnp.float32),
            pltpu.VMEM((WSZ,), jnp.int32),
            pltpu.VMEM((ROWS_CAP // 128, 128), jnp.int32),
            pltpu.SemaphoreType.DMA,
            pltpu.SemaphoreType.DMA,
        ],
    )(functools.partial(_fold_body, nv, plo_t, phi_t))


_BLK = 1000


def _mm_body(x_ref, w_ref, o_ref):
    o_ref[...] = jnp.dot(x_ref[...], w_ref[...],
                         preferred_element_type=jnp.float32)


def _mm(x, w):
    m, k = x.shape
    n = w.shape[1]
    return pl.pallas_call(
        _mm_body,
        grid=(m // _BLK,),
        in_specs=[pl.BlockSpec((_BLK, k), lambda i: (i, 0)),
                  pl.BlockSpec((k, n), lambda i: (0, 0))],
        out_specs=pl.BlockSpec((_BLK, n), lambda i: (i, 0)),
        out_shape=jax.ShapeDtypeStruct((m, n), jnp.float32),
    )(x, w)


def _fix_tanh_body(p_ref, side_ref, bias_ref, rfix_ref, valid_ref, o_ref):
    i = pl.program_id(0)
    rows = lax.broadcasted_iota(jnp.int32, (_BLK, 1), 0) + i * _BLK
    x = p_ref[...]
    for k in range(15):
        m = (rows == rfix_ref[k]) & (valid_ref[k] > 0)
        x = jnp.where(m, x + side_ref[k, :][None, :], x)
    o_ref[...] = jnp.tanh(x + bias_ref[...])


def _fix_tanh(p_full, bias, rfix, valid, dpad):
    return pl.pallas_call(
        _fix_tanh_body,
        grid=(N // _BLK,),
        in_specs=[
            pl.BlockSpec((_BLK, dpad), lambda i: (i, 0)),
            pl.BlockSpec((NSIDE, dpad), lambda i: ((N + NDUMP) // NSIDE, 0)),
            pl.BlockSpec((1, dpad), lambda i: (0, 0)),
            pl.BlockSpec(memory_space=pltpu.SMEM),
            pl.BlockSpec(memory_space=pltpu.SMEM),
        ],
        out_specs=pl.BlockSpec((_BLK, dpad), lambda i: (i, 0)),
        out_shape=jax.ShapeDtypeStruct((N, dpad), jnp.float32),
    )(p_full, p_full, bias.reshape(1, dpad), rfix, valid)


def _head_body(h_ref, w1_ref, b1_ref, w2_ref, b2_ref, wf_ref, bf_ref, o_ref):
    h = h_ref[...]
    h = jax.nn.relu(jnp.dot(h, w1_ref[...],
                            preferred_element_type=jnp.float32) + b1_ref[...])
    h = jax.nn.relu(jnp.dot(h, w2_ref[...],
                            preferred_element_type=jnp.float32) + b2_ref[...])
    o_ref[...] = jnp.tanh(jnp.dot(h, wf_ref[...],
                                  preferred_element_type=jnp.float32)
                          + bf_ref[...])


def _head(h, w1, b1, w2, b2, wf, bf):
    return pl.pallas_call(
        _head_body,
        grid=(N // _BLK,),
        in_specs=[
            pl.BlockSpec((_BLK, h.shape[1]), lambda i: (i, 0)),
            pl.BlockSpec(w1.shape, lambda i: (0, 0)),
            pl.BlockSpec((1, w1.shape[1]), lambda i: (0, 0)),
            pl.BlockSpec(w2.shape, lambda i: (0, 0)),
            pl.BlockSpec((1, w2.shape[1]), lambda i: (0, 0)),
            pl.BlockSpec(wf.shape, lambda i: (0, 0)),
            pl.BlockSpec((1, 1), lambda i: (0, 0)),
        ],
        out_specs=pl.BlockSpec((_BLK, 1), lambda i: (i, 0)),
        out_shape=jax.ShapeDtypeStruct((N, 1), jnp.float32),
    )(h, w1, b1.reshape(1, -1), w2, b2.reshape(1, -1), wf, bf.reshape(1, 1))


def _emb_body(x_ref, w_ref, b_ref, o_ref):
    o_ref[...] = jnp.dot(x_ref[...], w_ref[...],
                         preferred_element_type=jnp.float32) + b_ref[...]


def _emb(x, w, b):
    return pl.pallas_call(
        _emb_body,
        grid=(N // _BLK,),
        in_specs=[pl.BlockSpec((_BLK, 128), lambda i: (i, 0)),
                  pl.BlockSpec((128, 128), lambda i: (0, 0)),
                  pl.BlockSpec((1, 128), lambda i: (0, 0))],
        out_specs=pl.BlockSpec((_BLK, 128), lambda i: (i, 0)),
        out_shape=jax.ShapeDtypeStruct((N, 128), jnp.float32),
    )(x, w, b.reshape(1, 128))


def kernel(x, edge_index, edge_weight, emb_w, emb_b, gcn_ws, gcn_bs,
           mlp_ws, mlp_bs, final_w, final_b):
    row, col = edge_index[0], edge_index[1]

    x_emb = _emb(x, emb_w, emb_b)

    deg = jax.ops.segment_sum(edge_weight, row, num_segments=N)
    dinv = jnp.where(deg > 0, jax.lax.rsqrt(jnp.maximum(deg, 1e-12)), 0.0)
    w_norm = dinv[row] * edge_weight * dinv[col]

    loops = jnp.arange(N, dtype=edge_index.dtype)
    src_full = jnp.concatenate([row, loops])
    dst_full = jnp.concatenate([col, loops])
    ew_full = jnp.concatenate(
        [DELTA * w_norm, (1.0 - DELTA) * jnp.ones((N,), jnp.float32)])

    prep32 = _prep(dst_full, src_full, ew_full, 32)
    prep16 = _prep(dst_full, src_full, ew_full, 16)

    h = x_emb
    for li, (w, b) in enumerate(zip(gcn_ws, gcn_bs)):
        fo = w.shape[1]
        dpad = 32 if fo == 32 else 16
        p = prep32 if dpad == 32 else prep16
        w_pad = jnp.pad(w, ((0, 0), (0, dpad - fo)))
        b_pad = jnp.pad(b, (0, dpad - fo))
        hw = _mm(h, w_pad)
        (fold_out,) = _make_fold(dpad)(
            hw, p["srcs"], p["ews"], p["keep"], p["locs"], p["own_dst"])
        h_full = _fix_tanh(fold_out, b_pad, p["rfix"], p["valid"], dpad)
        h = h_full[:, :fo]

    score = _head(h, mlp_ws[0], mlp_bs[0], mlp_ws[1], mlp_bs[1],
                  final_w, final_b)

    zs = score[:, 0]
    num = jnp.sum(edge_weight * zs[row] * zs[col])
    den = jnp.sum(edge_weight) + 1e-12
    aux_loss = num / den

    _, kept_nodes = jax.lax.top_k(zs, K_KEEP)
    return x_emb, kept_nodes, zs, aux_loss


# speedup vs baseline: 41.2988x; 1.4803x over previous
import functools
import jax
import jax.numpy as jnp
from jax import lax
from jax.experimental import pallas as pl
from jax.experimental.pallas import tpu as pltpu, tpu_sc as plsc

N = 10000
E = 320000
E2 = E + N
DELTA = 2.0
K_KEEP = 5000

WSZ = 384
OV = 128
ROWS_CAP = 1024
NW = 32
NDUMP = 64
NSIDE = 16
NOUT = N + NDUMP + NSIDE

_B_TAIL = {32: [269184, 289536, 309888], 16: [269136, 289440, 309744]}


def _boundaries(dpad):
    return [20736 * k for k in range(1, 13)] + _B_TAIL[dpad]


def _chunk_tables(dpad):
    bf = [0] + _boundaries(dpad) + [E2]
    plo, phi, mids = [], [], []
    for c in range(16):
        lo, hi = bf[c], bf[c + 1]
        mid = lo + (hi - lo) // 2
        mids.append(mid)
        plo += [lo, mid - OV]
        phi += [mid, hi]
    sbase = [max(0, (N * bf[c]) // E2 - 64) for c in range(16)]
    return bf, plo, phi, mids, sbase


def _prep(ds, sorted_common, dpad):
    bf, _, _, mids, sbase = _chunk_tables(dpad)
    bnds = jnp.asarray(bf[1:-1], jnp.int32)
    chunk_of = jnp.searchsorted(bnds, jnp.arange(E2, dtype=jnp.int32),
                                side="right")
    sbase_arr = jnp.asarray(sbase, jnp.int32)
    locs = ds - sbase_arr[chunk_of]
    own_rows = []
    rfix = []
    valid = []
    slot_ids = jnp.arange(ROWS_CAP, dtype=jnp.int32)
    dump = (N + (slot_ids % NDUMP)).astype(jnp.int32)
    for c in range(16):
        lo, hi, mid, sb = bf[c], bf[c + 1], mids[c], sbase[c]
        r_lo, r_mid, r_midp, r_hi = ds[lo], ds[mid], ds[mid - 1], ds[hi - 1]
        slot_rows = sb + slot_ids
        top0 = r_midp - (r_midp == r_mid)
        d0 = jnp.where((slot_rows >= r_lo) & (slot_rows <= top0),
                       slot_rows, dump)
        own_rows.append(d0)
        if c < 15:
            strad = ds[hi - 1] == ds[hi]
            rfix.append(ds[hi])
            valid.append(strad.astype(jnp.float32))
        else:
            strad = jnp.zeros((), bool)
        top1 = r_hi - strad
        d1 = jnp.where((slot_rows >= r_mid) & (slot_rows <= top1),
                       slot_rows, dump)
        d1 = jnp.where((slot_rows == r_hi) & strad,
                       jnp.int32(N + NDUMP + c), d1)
        own_rows.append(d1)
    own_dst = jnp.stack(own_rows).reshape(NW, ROWS_CAP // 128, 128)
    pad_i = jnp.zeros((WSZ + 8,), jnp.int32)
    return dict(
        locs=jnp.concatenate([locs, pad_i]),
        own_dst=own_dst,
        rfix=jnp.stack(rfix + [jnp.int32(-1)]),
        valid=jnp.stack(valid + [jnp.float32(0.0)]),
        **sorted_common)


def _prep_common(dst_full, src_full, ew_full):
    perm = jnp.argsort(dst_full, stable=True).astype(jnp.int32)
    ds = dst_full[perm].astype(jnp.int32)
    srcs = src_full[perm].astype(jnp.int32)
    ews = ew_full[perm].astype(jnp.float32)
    chg = jnp.concatenate([jnp.ones((1,), bool), ds[1:] != ds[:-1]])
    keep = jnp.where(chg, 0.0, 1.0).astype(jnp.float32)
    pad_i = jnp.zeros((WSZ + 8,), jnp.int32)
    pad_f = jnp.zeros((WSZ + 8,), jnp.float32)
    common = dict(
        srcs=jnp.concatenate([srcs, pad_i]),
        ews=jnp.concatenate([ews, pad_f]),
        keep=jnp.concatenate([keep, pad_f]))
    return ds, common


def _psel(w, table):
    v = jnp.int32(table[NW - 1])
    for i in range(NW - 2, -1, -1):
        v = jnp.where(w == i, jnp.int32(table[i]), v)
    return v


def _fold_body(nv, plo_t, phi_t,
               hw, srcs, ews, keepf, locs, own_dst,
               out,
               idx_v, rows_v, out_local, ew_v, kp_v, lc_v, dst_v,
               sem_g, sem_s):
    c = lax.axis_index("c")
    s = lax.axis_index("s")
    w = s * 2 + c
    plo = _psel(w, plo_t)
    phi = _psel(w, phi_t)
    skip = lax.rem(plo, 8)
    abase = plo - skip
    ne = phi - plo
    nwin = lax.div(skip + ne + WSZ - 1, WSZ)

    zero = jnp.zeros((16,), jnp.float32)
    cols = [lax.broadcasted_iota(jnp.int32, (16,), 0) + t * 16
            for t in range(nv)]

    def window(g, acc):
        base = pl.multiple_of(abase + g * WSZ, 8)
        pltpu.sync_copy(srcs.at[pl.ds(base, WSZ)], idx_v)
        pltpu.async_copy(hw.at[idx_v], rows_v, sem_g).wait()
        pltpu.sync_copy(ews.at[pl.ds(base, WSZ)], ew_v)
        pltpu.sync_copy(keepf.at[pl.ds(base, WSZ)], kp_v)
        pltpu.sync_copy(locs.at[pl.ds(base, WSZ)], lc_v)
        j_lo = jnp.where(g == 0, skip, 0)
        j_hi = jnp.minimum(WSZ, skip + ne - g * WSZ)

        def body(j, a):
            jv = jnp.full((16,), j, jnp.int32)
            vew = plsc.load_gather(ew_v, [jv])
            vkp = plsc.load_gather(kp_v, [jv])
            vlc = plsc.load_gather(lc_v, [jv])
            na = []
            for t in range(nv):
                rowt = rows_v[j, pl.ds(t * 16, 16)]
                at = a[t] * vkp + vew * rowt
                plsc.store_scatter(out_local, [vlc, cols[t]], at)
                na.append(at)
            return tuple(na)

        return lax.fori_loop(j_lo, j_hi, body, acc)

    lax.fori_loop(0, nwin, window, (zero,) * nv)

    pltpu.sync_copy(own_dst.at[w], dst_v)
    for b in range(ROWS_CAP // 128):
        pltpu.async_copy(out_local.at[pl.ds(b * 128, 128)],
                         out.at[dst_v.at[b]], sem_s).wait()


@functools.cache
def _make_fold(dpad):
    nv = dpad // 16
    _, plo_t, phi_t, _, _ = _chunk_tables(dpad)
    mesh = plsc.VectorSubcoreMesh(core_axis_name="c", subcore_axis_name="s")
    return functools.partial(
        pl.kernel,
        mesh=mesh,
        compiler_params=pltpu.CompilerParams(use_tc_tiling_on_sc=False,
                                             needs_layout_passes=False),
        out_type=[jax.ShapeDtypeStruct((NOUT, dpad), jnp.float32)],
        scratch_types=[
            pltpu.VMEM((WSZ,), jnp.int32),
            pltpu.VMEM((WSZ, dpad), jnp.float32),
            pltpu.VMEM((ROWS_CAP, dpad), jnp.float32),
            pltpu.VMEM((WSZ,), jnp.float32),
            pltpu.VMEM((WSZ,), jnp.float32),
            pltpu.VMEM((WSZ,), jnp.int32),
            pltpu.VMEM((ROWS_CAP // 128, 128), jnp.int32),
            pltpu.SemaphoreType.DMA,
            pltpu.SemaphoreType.DMA,
        ],
    )(functools.partial(_fold_body, nv, plo_t, phi_t))


_BLK = 1000


def _mm_body(x_ref, w_ref, o_ref):
    o_ref[...] = jnp.dot(x_ref[...], w_ref[...],
                         preferred_element_type=jnp.float32)


def _mm(x, w):
    m, k = x.shape
    n = w.shape[1]
    return pl.pallas_call(
        _mm_body,
        grid=(m // _BLK,),
        in_specs=[pl.BlockSpec((_BLK, k), lambda i: (i, 0)),
                  pl.BlockSpec((k, n), lambda i: (0, 0))],
        out_specs=pl.BlockSpec((_BLK, n), lambda i: (i, 0)),
        out_shape=jax.ShapeDtypeStruct((m, n), jnp.float32),
    )(x, w)


def _fix_tanh_body(p_ref, side_ref, bias_ref, rfix_ref, valid_ref, o_ref):
    i = pl.program_id(0)
    rows = lax.broadcasted_iota(jnp.int32, (_BLK, 1), 0) + i * _BLK
    x = p_ref[...]
    for k in range(15):
        m = (rows == rfix_ref[k]) & (valid_ref[k] > 0)
        x = jnp.where(m, x + side_ref[k, :][None, :], x)
    o_ref[...] = jnp.tanh(x + bias_ref[...])


def _fix_tanh(p_full, bias, rfix, valid, dpad):
    return pl.pallas_call(
        _fix_tanh_body,
        grid=(N // _BLK,),
        in_specs=[
            pl.BlockSpec((_BLK, dpad), lambda i: (i, 0)),
            pl.BlockSpec((NSIDE, dpad), lambda i: ((N + NDUMP) // NSIDE, 0)),
            pl.BlockSpec((1, dpad), lambda i: (0, 0)),
            pl.BlockSpec(memory_space=pltpu.SMEM),
            pl.BlockSpec(memory_space=pltpu.SMEM),
        ],
        out_specs=pl.BlockSpec((_BLK, dpad), lambda i: (i, 0)),
        out_shape=jax.ShapeDtypeStruct((N, dpad), jnp.float32),
    )(p_full, p_full, bias.reshape(1, dpad), rfix, valid)


def _head_body(h_ref, w1_ref, b1_ref, w2_ref, b2_ref, wf_ref, bf_ref, o_ref):
    h = h_ref[...]
    h = jax.nn.relu(jnp.dot(h, w1_ref[...],
                            preferred_element_type=jnp.float32) + b1_ref[...])
    h = jax.nn.relu(jnp.dot(h, w2_ref[...],
                            preferred_element_type=jnp.float32) + b2_ref[...])
    o_ref[...] = jnp.tanh(jnp.dot(h, wf_ref[...],
                                  preferred_element_type=jnp.float32)
                          + bf_ref[...])


def _head(h, w1, b1, w2, b2, wf, bf):
    return pl.pallas_call(
        _head_body,
        grid=(N // _BLK,),
        in_specs=[
            pl.BlockSpec((_BLK, h.shape[1]), lambda i: (i, 0)),
            pl.BlockSpec(w1.shape, lambda i: (0, 0)),
            pl.BlockSpec((1, w1.shape[1]), lambda i: (0, 0)),
            pl.BlockSpec(w2.shape, lambda i: (0, 0)),
            pl.BlockSpec((1, w2.shape[1]), lambda i: (0, 0)),
            pl.BlockSpec(wf.shape, lambda i: (0, 0)),
            pl.BlockSpec((1, 1), lambda i: (0, 0)),
        ],
        out_specs=pl.BlockSpec((_BLK, 1), lambda i: (i, 0)),
        out_shape=jax.ShapeDtypeStruct((N, 1), jnp.float32),
    )(h, w1, b1.reshape(1, -1), w2, b2.reshape(1, -1), wf, bf.reshape(1, 1))


def _emb_body(x_ref, w_ref, b_ref, o_ref):
    o_ref[...] = jnp.dot(x_ref[...], w_ref[...],
                         preferred_element_type=jnp.float32) + b_ref[...]


def _emb(x, w, b):
    return pl.pallas_call(
        _emb_body,
        grid=(N // _BLK,),
        in_specs=[pl.BlockSpec((_BLK, 128), lambda i: (i, 0)),
                  pl.BlockSpec((128, 128), lambda i: (0, 0)),
                  pl.BlockSpec((1, 128), lambda i: (0, 0))],
        out_specs=pl.BlockSpec((_BLK, 128), lambda i: (i, 0)),
        out_shape=jax.ShapeDtypeStruct((N, 128), jnp.float32),
    )(x, w, b.reshape(1, 128))


def _rank_body(sb_ref, sa_ref, o_ref):
    i = pl.program_id(0)
    j = pl.program_id(1)
    sb = sb_ref[...]
    sa = sa_ref[...].reshape(1, _BLK)
    ib = lax.broadcasted_iota(jnp.int32, (_BLK, 1), 0) + i * _BLK
    ia = lax.broadcasted_iota(jnp.int32, (1, _BLK), 1) + j * _BLK
    part = jnp.sum(((sa > sb) | ((sa == sb) & (ia < ib))).astype(jnp.int32),
                   axis=1, keepdims=True)

    @pl.when(j == 0)
    def _():
        o_ref[...] = part

    @pl.when(j != 0)
    def _():
        o_ref[...] = o_ref[...] + part


def _rank(zs):
    s2 = zs.reshape(N, 1)
    s3 = zs.reshape(N // _BLK, 1, _BLK)
    return pl.pallas_call(
        _rank_body,
        grid=(N // _BLK, N // _BLK),
        in_specs=[pl.BlockSpec((_BLK, 1), lambda i, j: (i, 0)),
                  pl.BlockSpec((1, 1, _BLK), lambda i, j: (j, 0, 0))],
        out_specs=pl.BlockSpec((_BLK, 1), lambda i, j: (i, 0)),
        out_shape=jax.ShapeDtypeStruct((N, 1), jnp.int32),
    )(s2, s3)[:, 0]


def _select_body(rank_hbm, kept_out, rank_v, kept_v, ids0, sem):
    c = lax.axis_index("c")
    s = lax.axis_index("s")
    w = s * 2 + c

    @pl.when(w == 0)
    def _():
        pltpu.sync_copy(rank_hbm, rank_v)

        def blk(b, _):
            r16 = rank_v[pl.ds(pl.multiple_of(b * 16, 16), 16)]
            ids = ids0 + b * 16
            plsc.store_scatter(kept_v, [r16], ids, mask=r16 < K_KEEP)
            return 0

        lax.fori_loop(0, N // 16, blk, 0)
        pltpu.sync_copy(kept_v, kept_out)


@functools.cache
def _make_select():
    mesh = plsc.VectorSubcoreMesh(core_axis_name="c", subcore_axis_name="s")

    def body(rank_hbm, kept_out, rank_v, kept_v, sem):
        ids0 = lax.broadcasted_iota(jnp.int32, (16,), 0)
        _select_body(rank_hbm, kept_out, rank_v, kept_v, ids0, sem)

    return functools.partial(
        pl.kernel,
        mesh=mesh,
        compiler_params=pltpu.CompilerParams(use_tc_tiling_on_sc=False,
                                             needs_layout_passes=False),
        out_type=[jax.ShapeDtypeStruct((K_KEEP,), jnp.int32)],
        scratch_types=[
            pltpu.VMEM((N,), jnp.int32),
            pltpu.VMEM((K_KEEP,), jnp.int32),
            pltpu.SemaphoreType.DMA,
        ],
    )(body)


_LW = 400


def _loss_body(rows, cols, ews, zs, out,
               ridx_v, zr_v, zc_v, ew_v, acc_v, sem):
    c = lax.axis_index("c")
    s = lax.axis_index("s")
    w = s * 2 + c
    base0 = w * (E // NW)

    def window(g, acc):
        base = pl.multiple_of(base0 + g * _LW, 8)
        pltpu.sync_copy(rows.at[pl.ds(base, _LW)], ridx_v)
        pltpu.async_copy(zs.at[ridx_v], zr_v, sem).wait()
        pltpu.sync_copy(cols.at[pl.ds(base, _LW)], ridx_v)
        pltpu.async_copy(zs.at[ridx_v], zc_v, sem).wait()
        pltpu.sync_copy(ews.at[pl.ds(base, _LW)], ew_v)

        def body(k, a):
            sl = pl.ds(pl.multiple_of(k * 16, 16), 16)
            ew16 = ew_v[sl]
            return (a[0] + ew16 * zr_v[sl] * zc_v[sl], a[1] + ew16)

        return lax.fori_loop(0, _LW // 16, body, acc)

    z16 = jnp.zeros((16,), jnp.float32)
    accn, accd = lax.fori_loop(0, (E // NW) // _LW, window, (z16, z16))
    acc_v[pl.ds(0, 16)] = accn
    acc_v[pl.ds(16, 16)] = accd
    pltpu.sync_copy(acc_v, out.at[w])


@functools.cache
def _make_loss():
    mesh = plsc.VectorSubcoreMesh(core_axis_name="c", subcore_axis_name="s")
    return functools.partial(
        pl.kernel,
        mesh=mesh,
        compiler_params=pltpu.CompilerParams(use_tc_tiling_on_sc=False,
                                             needs_layout_passes=False),
        out_type=[jax.ShapeDtypeStruct((NW, 32), jnp.float32)],
        scratch_types=[
            pltpu.VMEM((_LW,), jnp.int32),
            pltpu.VMEM((_LW,), jnp.float32),
            pltpu.VMEM((_LW,), jnp.float32),
            pltpu.VMEM((_LW,), jnp.float32),
            pltpu.VMEM((32,), jnp.float32),
            pltpu.SemaphoreType.DMA,
        ],
    )(_loss_body)


def kernel(x, edge_index, edge_weight, emb_w, emb_b, gcn_ws, gcn_bs,
           mlp_ws, mlp_bs, final_w, final_b):
    row, col = edge_index[0], edge_index[1]

    x_emb = _emb(x, emb_w, emb_b)

    deg = jax.ops.segment_sum(edge_weight, row, num_segments=N)
    dinv = jnp.where(deg > 0, jax.lax.rsqrt(jnp.maximum(deg, 1e-12)), 0.0)
    w_norm = dinv[row] * edge_weight * dinv[col]

    loops = jnp.arange(N, dtype=edge_index.dtype)
    src_full = jnp.concatenate([row, loops])
    dst_full = jnp.concatenate([col, loops])
    ew_full = jnp.concatenate(
        [DELTA * w_norm, (1.0 - DELTA) * jnp.ones((N,), jnp.float32)])

    ds, common = _prep_common(dst_full, src_full, ew_full)
    prep32 = _prep(ds, common, 32)
    prep16 = _prep(ds, common, 16)

    h = x_emb
    for li, (w, b) in enumerate(zip(gcn_ws, gcn_bs)):
        fo = w.shape[1]
        dpad = 32 if fo == 32 else 16
        p = prep32 if dpad == 32 else prep16
        w_pad = jnp.pad(w, ((0, 0), (0, dpad - fo)))
        b_pad = jnp.pad(b, (0, dpad - fo))
        hw = _mm(h, w_pad)
        (fold_out,) = _make_fold(dpad)(
            hw, p["srcs"], p["ews"], p["keep"], p["locs"], p["own_dst"])
        h_full = _fix_tanh(fold_out, b_pad, p["rfix"], p["valid"], dpad)
        h = h_full[:, :fo]

    score = _head(h, mlp_ws[0], mlp_bs[0], mlp_ws[1], mlp_bs[1],
                  final_w, final_b)

    zs = score[:, 0]
    (loss_parts,) = _make_loss()(row, col, edge_weight, zs)
    num = jnp.sum(loss_parts[:, :16])
    den = jnp.sum(loss_parts[:, 16:]) + 1e-12
    aux_loss = num / den

    rank = _rank(zs)
    (kept_nodes,) = _make_select()(rank)
    return x_emb, kept_nodes, zs, aux_loss


# speedup vs baseline: 44.8677x; 1.0864x over previous
import functools
import jax
import jax.numpy as jnp
from jax import lax
from jax.experimental import pallas as pl
from jax.experimental.pallas import tpu as pltpu, tpu_sc as plsc

N = 10000
E = 320000
E2 = E + N
DELTA = 2.0
K_KEEP = 5000

WSZ = 1024
OV = 128
ROWS_CAP = 1024
NW = 32
NDUMP = 64
NSIDE = 16
NOUT = N + NDUMP + NSIDE

_B_TAIL = {32: [269184, 289536, 309888], 16: [269136, 289440, 309744]}


def _boundaries(dpad):
    return [20736 * k for k in range(1, 13)] + _B_TAIL[dpad]


def _chunk_tables(dpad):
    bf = [0] + _boundaries(dpad) + [E2]
    plo, phi, mids = [], [], []
    for c in range(16):
        lo, hi = bf[c], bf[c + 1]
        mid = lo + (hi - lo) // 2
        mids.append(mid)
        plo += [lo, mid - OV]
        phi += [mid, hi]
    sbase = [max(0, (N * bf[c]) // E2 - 64) for c in range(16)]
    return bf, plo, phi, mids, sbase


def _prep(ds, sorted_common, dpad):
    bf, _, _, mids, sbase = _chunk_tables(dpad)
    bnds = jnp.asarray(bf[1:-1], jnp.int32)
    chunk_of = jnp.searchsorted(bnds, jnp.arange(E2, dtype=jnp.int32),
                                side="right")
    sbase_arr = jnp.asarray(sbase, jnp.int32)
    locs = ds - sbase_arr[chunk_of]
    own_rows = []
    rfix = []
    valid = []
    slot_ids = jnp.arange(ROWS_CAP, dtype=jnp.int32)
    dump = (N + (slot_ids % NDUMP)).astype(jnp.int32)
    for c in range(16):
        lo, hi, mid, sb = bf[c], bf[c + 1], mids[c], sbase[c]
        r_lo, r_mid, r_midp, r_hi = ds[lo], ds[mid], ds[mid - 1], ds[hi - 1]
        slot_rows = sb + slot_ids
        top0 = r_midp - (r_midp == r_mid)
        d0 = jnp.where((slot_rows >= r_lo) & (slot_rows <= top0),
                       slot_rows, dump)
        own_rows.append(d0)
        if c < 15:
            strad = ds[hi - 1] == ds[hi]
            rfix.append(ds[hi])
            valid.append(strad.astype(jnp.float32))
        else:
            strad = jnp.zeros((), bool)
        top1 = r_hi - strad
        d1 = jnp.where((slot_rows >= r_mid) & (slot_rows <= top1),
                       slot_rows, dump)
        d1 = jnp.where((slot_rows == r_hi) & strad,
                       jnp.int32(N + NDUMP + c), d1)
        own_rows.append(d1)
    own_dst = jnp.stack(own_rows).reshape(NW, ROWS_CAP // 128, 128)
    pad_i = jnp.zeros((WSZ + 8,), jnp.int32)
    return dict(
        locs=jnp.concatenate([locs, pad_i]),
        own_dst=own_dst,
        rfix=jnp.stack(rfix + [jnp.int32(-1)]),
        valid=jnp.stack(valid + [jnp.float32(0.0)]),
        **sorted_common)


def _prep_common(dst_full, src_full, ew_full):
    perm = jnp.argsort(dst_full, stable=True).astype(jnp.int32)
    ds = dst_full[perm].astype(jnp.int32)
    srcs = src_full[perm].astype(jnp.int32)
    ews = ew_full[perm].astype(jnp.float32)
    chg = jnp.concatenate([jnp.ones((1,), bool), ds[1:] != ds[:-1]])
    keep = jnp.where(chg, 0.0, 1.0).astype(jnp.float32)
    pad_i = jnp.zeros((WSZ + 8,), jnp.int32)
    pad_f = jnp.zeros((WSZ + 8,), jnp.float32)
    common = dict(
        srcs=jnp.concatenate([srcs, pad_i]),
        ews=jnp.concatenate([ews, pad_f]),
        keep=jnp.concatenate([keep, pad_f]))
    return ds, common


def _psel(w, table):
    v = jnp.int32(table[NW - 1])
    for i in range(NW - 2, -1, -1):
        v = jnp.where(w == i, jnp.int32(table[i]), v)
    return v


def _fold_body(nv, plo_t, phi_t,
               hw, srcs, ews, keepf, locs, own_dst,
               out,
               idx_v, rows_v, out_local, ew_v, kp_v, lc_v, dst_v,
               sem_g, sem_s):
    c = lax.axis_index("c")
    s = lax.axis_index("s")
    w = s * 2 + c
    plo = _psel(w, plo_t)
    phi = _psel(w, phi_t)
    skip = lax.rem(plo, 8)
    abase = plo - skip
    ne = phi - plo
    nwin = lax.div(skip + ne + WSZ - 1, WSZ)

    zero = jnp.zeros((16,), jnp.float32)
    cols = [lax.broadcasted_iota(jnp.int32, (16,), 0) + t * 16
            for t in range(nv)]

    def window(g, acc):
        base = pl.multiple_of(abase + g * WSZ, 8)
        pltpu.sync_copy(srcs.at[pl.ds(base, WSZ)], idx_v)
        gather = pltpu.async_copy(hw.at[idx_v], rows_v, sem_g)
        pltpu.sync_copy(ews.at[pl.ds(base, WSZ)], ew_v)
        pltpu.sync_copy(keepf.at[pl.ds(base, WSZ)], kp_v)
        pltpu.sync_copy(locs.at[pl.ds(base, WSZ)], lc_v)
        gather.wait()
        j_lo = jnp.where(g == 0, skip, 0)
        j_hi = jnp.minimum(WSZ, skip + ne - g * WSZ)

        def body(j, a):
            jv = jnp.full((16,), j, jnp.int32)
            vew = plsc.load_gather(ew_v, [jv])
            vkp = plsc.load_gather(kp_v, [jv])
            vlc = plsc.load_gather(lc_v, [jv])
            na = []
            for t in range(nv):
                rowt = rows_v[j, pl.ds(t * 16, 16)]
                at = a[t] * vkp + vew * rowt
                plsc.store_scatter(out_local, [vlc, cols[t]], at)
                na.append(at)
            return tuple(na)

        return lax.fori_loop(j_lo, j_hi, body, acc)

    lax.fori_loop(0, nwin, window, (zero,) * nv)

    pltpu.sync_copy(own_dst.at[w], dst_v)
    for b in range(ROWS_CAP // 128):
        pltpu.async_copy(out_local.at[pl.ds(b * 128, 128)],
                         out.at[dst_v.at[b]], sem_s).wait()


@functools.cache
def _make_fold(dpad):
    nv = dpad // 16
    _, plo_t, phi_t, _, _ = _chunk_tables(dpad)
    mesh = plsc.VectorSubcoreMesh(core_axis_name="c", subcore_axis_name="s")
    return functools.partial(
        pl.kernel,
        mesh=mesh,
        compiler_params=pltpu.CompilerParams(use_tc_tiling_on_sc=False,
                                             needs_layout_passes=False),
        out_type=[jax.ShapeDtypeStruct((NOUT, dpad), jnp.float32)],
        scratch_types=[
            pltpu.VMEM((WSZ,), jnp.int32),
            pltpu.VMEM((WSZ, dpad), jnp.float32),
            pltpu.VMEM((ROWS_CAP, dpad), jnp.float32),
            pltpu.VMEM((WSZ,), jnp.float32),
            pltpu.VMEM((WSZ,), jnp.float32),
            pltpu.VMEM((WSZ,), jnp.int32),
            pltpu.VMEM((ROWS_CAP // 128, 128), jnp.int32),
            pltpu.SemaphoreType.DMA,
            pltpu.SemaphoreType.DMA,
        ],
    )(functools.partial(_fold_body, nv, plo_t, phi_t))


_BLK = 1000


def _mm_body(x_ref, w_ref, o_ref):
    o_ref[...] = jnp.dot(x_ref[...], w_ref[...],
                         preferred_element_type=jnp.float32)


def _mm(x, w):
    m, k = x.shape
    n = w.shape[1]
    return pl.pallas_call(
        _mm_body,
        grid=(m // _BLK,),
        in_specs=[pl.BlockSpec((_BLK, k), lambda i: (i, 0)),
                  pl.BlockSpec((k, n), lambda i: (0, 0))],
        out_specs=pl.BlockSpec((_BLK, n), lambda i: (i, 0)),
        out_shape=jax.ShapeDtypeStruct((m, n), jnp.float32),
    )(x, w)


def _fix_tanh_body(p_ref, side_ref, bias_ref, rfix_ref, valid_ref, o_ref):
    i = pl.program_id(0)
    rows = lax.broadcasted_iota(jnp.int32, (_BLK, 1), 0) + i * _BLK
    x = p_ref[...]
    for k in range(15):
        m = (rows == rfix_ref[k]) & (valid_ref[k] > 0)
        x = jnp.where(m, x + side_ref[k, :][None, :], x)
    o_ref[...] = jnp.tanh(x + bias_ref[...])


def _fix_tanh(p_full, bias, rfix, valid, dpad):
    return pl.pallas_call(
        _fix_tanh_body,
        grid=(N // _BLK,),
        in_specs=[
            pl.BlockSpec((_BLK, dpad), lambda i: (i, 0)),
            pl.BlockSpec((NSIDE, dpad), lambda i: ((N + NDUMP) // NSIDE, 0)),
            pl.BlockSpec((1, dpad), lambda i: (0, 0)),
            pl.BlockSpec(memory_space=pltpu.SMEM),
            pl.BlockSpec(memory_space=pltpu.SMEM),
        ],
        out_specs=pl.BlockSpec((_BLK, dpad), lambda i: (i, 0)),
        out_shape=jax.ShapeDtypeStruct((N, dpad), jnp.float32),
    )(p_full, p_full, bias.reshape(1, dpad), rfix, valid)


def _head_body(h_ref, w1_ref, b1_ref, w2_ref, b2_ref, wf_ref, bf_ref, o_ref):
    h = h_ref[...]
    h = jax.nn.relu(jnp.dot(h, w1_ref[...],
                            preferred_element_type=jnp.float32) + b1_ref[...])
    h = jax.nn.relu(jnp.dot(h, w2_ref[...],
                            preferred_element_type=jnp.float32) + b2_ref[...])
    o_ref[...] = jnp.tanh(jnp.dot(h, wf_ref[...],
                                  preferred_element_type=jnp.float32)
                          + bf_ref[...])


def _head(h, w1, b1, w2, b2, wf, bf):
    return pl.pallas_call(
        _head_body,
        grid=(N // _BLK,),
        in_specs=[
            pl.BlockSpec((_BLK, h.shape[1]), lambda i: (i, 0)),
            pl.BlockSpec(w1.shape, lambda i: (0, 0)),
            pl.BlockSpec((1, w1.shape[1]), lambda i: (0, 0)),
            pl.BlockSpec(w2.shape, lambda i: (0, 0)),
            pl.BlockSpec((1, w2.shape[1]), lambda i: (0, 0)),
            pl.BlockSpec(wf.shape, lambda i: (0, 0)),
            pl.BlockSpec((1, 1), lambda i: (0, 0)),
        ],
        out_specs=pl.BlockSpec((_BLK, 1), lambda i: (i, 0)),
        out_shape=jax.ShapeDtypeStruct((N, 1), jnp.float32),
    )(h, w1, b1.reshape(1, -1), w2, b2.reshape(1, -1), wf, bf.reshape(1, 1))


def _emb_body(x_ref, w_ref, b_ref, o_ref):
    o_ref[...] = jnp.dot(x_ref[...], w_ref[...],
                         preferred_element_type=jnp.float32) + b_ref[...]


def _emb(x, w, b):
    return pl.pallas_call(
        _emb_body,
        grid=(N // _BLK,),
        in_specs=[pl.BlockSpec((_BLK, 128), lambda i: (i, 0)),
                  pl.BlockSpec((128, 128), lambda i: (0, 0)),
                  pl.BlockSpec((1, 128), lambda i: (0, 0))],
        out_specs=pl.BlockSpec((_BLK, 128), lambda i: (i, 0)),
        out_shape=jax.ShapeDtypeStruct((N, 128), jnp.float32),
    )(x, w, b.reshape(1, 128))


def _rank_body(sb_ref, sa_ref, o_ref):
    i = pl.program_id(0)
    j = pl.program_id(1)
    sb = sb_ref[...]
    sa = sa_ref[...].reshape(1, _BLK)
    ib = lax.broadcasted_iota(jnp.int32, (_BLK, 1), 0) + i * _BLK
    ia = lax.broadcasted_iota(jnp.int32, (1, _BLK), 1) + j * _BLK
    part = jnp.sum(((sa > sb) | ((sa == sb) & (ia < ib))).astype(jnp.int32),
                   axis=1, keepdims=True)

    @pl.when(j == 0)
    def _():
        o_ref[...] = part

    @pl.when(j != 0)
    def _():
        o_ref[...] = o_ref[...] + part


def _rank(zs):
    s2 = zs.reshape(N, 1)
    s3 = zs.reshape(N // _BLK, 1, _BLK)
    return pl.pallas_call(
        _rank_body,
        grid=(N // _BLK, N // _BLK),
        in_specs=[pl.BlockSpec((_BLK, 1), lambda i, j: (i, 0)),
                  pl.BlockSpec((1, 1, _BLK), lambda i, j: (j, 0, 0))],
        out_specs=pl.BlockSpec((_BLK, 1), lambda i, j: (i, 0)),
        out_shape=jax.ShapeDtypeStruct((N, 1), jnp.int32),
    )(s2, s3)[:, 0]


def _select_body(rank_hbm, kept_out, rank_v, kept_v, ids0, sem):
    c = lax.axis_index("c")
    s = lax.axis_index("s")
    w = s * 2 + c

    @pl.when(w == 0)
    def _():
        pltpu.sync_copy(rank_hbm, rank_v)

        def blk(b, _):
            r16 = rank_v[pl.ds(pl.multiple_of(b * 16, 16), 16)]
            ids = ids0 + b * 16
            plsc.store_scatter(kept_v, [r16], ids, mask=r16 < K_KEEP)
            return 0

        lax.fori_loop(0, N // 16, blk, 0)
        pltpu.sync_copy(kept_v, kept_out)


@functools.cache
def _make_select():
    mesh = plsc.VectorSubcoreMesh(core_axis_name="c", subcore_axis_name="s")

    def body(rank_hbm, kept_out, rank_v, kept_v, sem):
        ids0 = lax.broadcasted_iota(jnp.int32, (16,), 0)
        _select_body(rank_hbm, kept_out, rank_v, kept_v, ids0, sem)

    return functools.partial(
        pl.kernel,
        mesh=mesh,
        compiler_params=pltpu.CompilerParams(use_tc_tiling_on_sc=False,
                                             needs_layout_passes=False),
        out_type=[jax.ShapeDtypeStruct((K_KEEP,), jnp.int32)],
        scratch_types=[
            pltpu.VMEM((N,), jnp.int32),
            pltpu.VMEM((K_KEEP,), jnp.int32),
            pltpu.SemaphoreType.DMA,
        ],
    )(body)


_LW = 400


def _loss_body(rows, cols, ews, zs, out,
               ridx_v, zr_v, zc_v, ew_v, acc_v, sem):
    c = lax.axis_index("c")
    s = lax.axis_index("s")
    w = s * 2 + c
    base0 = w * (E // NW)

    def window(g, acc):
        base = pl.multiple_of(base0 + g * _LW, 8)
        pltpu.sync_copy(rows.at[pl.ds(base, _LW)], ridx_v)
        pltpu.async_copy(zs.at[ridx_v], zr_v, sem).wait()
        pltpu.sync_copy(cols.at[pl.ds(base, _LW)], ridx_v)
        pltpu.async_copy(zs.at[ridx_v], zc_v, sem).wait()
        pltpu.sync_copy(ews.at[pl.ds(base, _LW)], ew_v)

        def body(k, a):
            sl = pl.ds(pl.multiple_of(k * 16, 16), 16)
            ew16 = ew_v[sl]
            return (a[0] + ew16 * zr_v[sl] * zc_v[sl], a[1] + ew16)

        return lax.fori_loop(0, _LW // 16, body, acc)

    z16 = jnp.zeros((16,), jnp.float32)
    accn, accd = lax.fori_loop(0, (E // NW) // _LW, window, (z16, z16))
    acc_v[pl.ds(0, 16)] = accn
    acc_v[pl.ds(16, 16)] = accd
    pltpu.sync_copy(acc_v, out.at[w])


@functools.cache
def _make_loss():
    mesh = plsc.VectorSubcoreMesh(core_axis_name="c", subcore_axis_name="s")
    return functools.partial(
        pl.kernel,
        mesh=mesh,
        compiler_params=pltpu.CompilerParams(use_tc_tiling_on_sc=False,
                                             needs_layout_passes=False),
        out_type=[jax.ShapeDtypeStruct((NW, 32), jnp.float32)],
        scratch_types=[
            pltpu.VMEM((_LW,), jnp.int32),
            pltpu.VMEM((_LW,), jnp.float32),
            pltpu.VMEM((_LW,), jnp.float32),
            pltpu.VMEM((_LW,), jnp.float32),
            pltpu.VMEM((32,), jnp.float32),
            pltpu.SemaphoreType.DMA,
        ],
    )(_loss_body)


def kernel(x, edge_index, edge_weight, emb_w, emb_b, gcn_ws, gcn_bs,
           mlp_ws, mlp_bs, final_w, final_b):
    row, col = edge_index[0], edge_index[1]

    x_emb = _emb(x, emb_w, emb_b)

    deg = jax.ops.segment_sum(edge_weight, row, num_segments=N)
    dinv = jnp.where(deg > 0, jax.lax.rsqrt(jnp.maximum(deg, 1e-12)), 0.0)
    w_norm = dinv[row] * edge_weight * dinv[col]

    loops = jnp.arange(N, dtype=edge_index.dtype)
    src_full = jnp.concatenate([row, loops])
    dst_full = jnp.concatenate([col, loops])
    ew_full = jnp.concatenate(
        [DELTA * w_norm, (1.0 - DELTA) * jnp.ones((N,), jnp.float32)])

    ds, common = _prep_common(dst_full, src_full, ew_full)
    prep32 = _prep(ds, common, 32)
    prep16 = _prep(ds, common, 16)

    h = x_emb
    for li, (w, b) in enumerate(zip(gcn_ws, gcn_bs)):
        fo = w.shape[1]
        dpad = 32 if fo == 32 else 16
        p = prep32 if dpad == 32 else prep16
        w_pad = jnp.pad(w, ((0, 0), (0, dpad - fo)))
        b_pad = jnp.pad(b, (0, dpad - fo))
        hw = _mm(h, w_pad)
        (fold_out,) = _make_fold(dpad)(
            hw, p["srcs"], p["ews"], p["keep"], p["locs"], p["own_dst"])
        h_full = _fix_tanh(fold_out, b_pad, p["rfix"], p["valid"], dpad)
        h = h_full[:, :fo]

    score = _head(h, mlp_ws[0], mlp_bs[0], mlp_ws[1], mlp_bs[1],
                  final_w, final_b)

    zs = score[:, 0]
    (loss_parts,) = _make_loss()(row, col, edge_weight, zs)
    num = jnp.sum(loss_parts[:, :16])
    den = jnp.sum(loss_parts[:, 16:]) + 1e-12
    aux_loss = num / den

    rank = _rank(zs)
    (kept_nodes,) = _make_select()(rank)
    return x_emb, kept_nodes, zs, aux_loss
